# baseline (speedup 1.0000x reference)
import functools

import jax
import jax.numpy as jnp
from jax import lax
from jax.experimental import pallas as pl
from jax.experimental.pallas import tpu as pltpu
from jax.experimental.pallas import tpu_sc as plsc

_N, _E, _D, _H = 10000, 160000, 256, 4
_HC = 1024
_CH = 40
_NW = 32
_EPW_B = _E // _NW
_NCH_B = _EPW_B // _CH
_EPT_C = _E // 16
_CHA = 40
_NCHA = _EPT_C // _CHA
_RPT = 624

_mesh = plsc.VectorSubcoreMesh(core_axis_name="c", subcore_axis_name="s")
_sc_params = pltpu.CompilerParams(use_tc_tiling_on_sc=False,
                                  needs_layout_passes=False)


def _proj_body(x_ref, w_ref, b_ref, q_ref, k_ref,
               v0_ref, v1_ref, v2_ref, v3_ref, sk_ref):
    y = jnp.dot(x_ref[...], w_ref[...], preferred_element_type=jnp.float32)
    y = y + b_ref[...]
    q_ref[...] = y[:, :1024].astype(jnp.bfloat16)
    k_ref[...] = y[:, 1024:2048].astype(jnp.bfloat16)
    v0_ref[...] = y[:, 2048:2304]
    v1_ref[...] = y[:, 2304:2560]
    v2_ref[...] = y[:, 2560:2816]
    v3_ref[...] = y[:, 2816:3072]
    sk_ref[...] = y[:, 3072:]


_proj = pl.pallas_call(
    _proj_body,
    grid=(10,),
    in_specs=[
        pl.BlockSpec((1000, _D), lambda i: (i, 0)),
        pl.BlockSpec((_D, 3328), lambda i: (0, 0)),
        pl.BlockSpec((1, 3328), lambda i: (0, 0)),
    ],
    out_specs=[
        pl.BlockSpec((1000, 1024), lambda i: (i, 0)),
        pl.BlockSpec((1000, 1024), lambda i: (i, 0)),
        pl.BlockSpec((1000, 256), lambda i: (i, 0)),
        pl.BlockSpec((1000, 256), lambda i: (i, 0)),
        pl.BlockSpec((1000, 256), lambda i: (i, 0)),
        pl.BlockSpec((1000, 256), lambda i: (i, 0)),
        pl.BlockSpec((1000, 256), lambda i: (i, 0)),
    ],
    out_shape=[
        jax.ShapeDtypeStruct((_N, 1024), jnp.bfloat16),
        jax.ShapeDtypeStruct((_N, 1024), jnp.bfloat16),
        jax.ShapeDtypeStruct((_N, 256), jnp.float32),
        jax.ShapeDtypeStruct((_N, 256), jnp.float32),
        jax.ShapeDtypeStruct((_N, 256), jnp.float32),
        jax.ShapeDtypeStruct((_N, 256), jnp.float32),
        jax.ShapeDtypeStruct((_N, 256), jnp.float32),
    ],
)


def _edge_body(q_hbm, k_hbm, dst2d_hbm, src2d_hbm,
               ex_hbm, d0_hbm, d1_hbm,
               dstv, srcv, qrows, krows, exstage, zbuf, accum, sem1, sem2):
    cid = lax.axis_index("c")
    sid = lax.axis_index("s")
    wid = sid * 2 + cid

    pltpu.sync_copy(dst2d_hbm.at[wid], dstv)
    pltpu.sync_copy(src2d_hbm.at[wid], srcv)

    zero16 = jnp.zeros((16,), jnp.float32)

    def _z(r, carry):
        zbuf[r] = zero16
        return carry
    lax.fori_loop(0, 640, _z, 0)

    @pl.when(sid < 15)
    def _():
        pltpu.sync_copy(zbuf.at[pl.ds(0, _RPT)], accum.at[pl.ds(sid * _RPT, _RPT)])

    @pl.when(sid == 15)
    def _():
        pltpu.sync_copy(zbuf, accum.at[pl.ds(15 * _RPT, 640)])
    plsc.subcore_barrier()

    lanes = lax.iota(jnp.int32, 16)

    dnums = lax.GatherDimensionNumbers(
        offset_dims=(), collapsed_slice_dims=(0,), start_index_map=(0,))

    def _lane_sum(a):
        for s in (8, 4, 2, 1):
            perm = lax.gather(a, (lanes ^ s)[:, None], dnums, (1,),
                              mode=lax.GatherScatterMode.PROMISE_IN_BOUNDS)
            a = a + perm
        return a

    mask_hi = jnp.full((16,), -65536, jnp.int32)

    def _bf2f32(x):
        hi = plsc.bitcast(x & mask_hi, jnp.float32)
        lo = plsc.bitcast(lax.shift_left(x, 16), jnp.float32)
        return lo, hi

    def _issue(ch):
        par = lax.rem(ch, 2)
        pltpu.async_copy(q_hbm.at[dstv.at[ch]], qrows.at[par], sem1.at[par])
        pltpu.async_copy(k_hbm.at[srcv.at[ch]], krows.at[par], sem2.at[par])

    _issue(0)

    def _chunk(ch, carry):
        par = lax.rem(ch, 2)
        pltpu.make_async_copy(q_hbm.at[dstv.at[ch]], qrows.at[par],
                              sem1.at[par]).wait()
        pltpu.make_async_copy(k_hbm.at[srcv.at[ch]], krows.at[par],
                              sem2.at[par]).wait()

        @pl.when(ch < _NCH_B - 1)
        def _():
            _issue(ch + 1)

        def _edge(i, c2):
            accs = []
            for h in range(_H):
                a = None
                for t in range(8):
                    off = h * 128 + t * 16
                    qw = qrows[par, i, pl.ds(off, 16)]
                    kw = krows[par, i, pl.ds(off, 16)]
                    qa, qb = _bf2f32(qw)
                    ka, kb = _bf2f32(kw)
                    p = qa * ka + qb * kb
                    a = p if a is None else a + p
                accs.append(_lane_sum(a))
            lv = jnp.where(lanes == 0, accs[0],
                           jnp.where(lanes == 1, accs[1],
                                     jnp.where(lanes == 2, accs[2], accs[3])))
            ex = jnp.exp(lv * 0.0625)
            ex = jnp.where(lanes < _H, ex, 0.0)
            exstage[i] = ex
            return c2
        lax.fori_loop(0, _CH, _edge, 0)

        pltpu.sync_copy(exstage, ex_hbm.at[pl.ds(wid * _EPW_B + ch * _CH, _CH)])
        pltpu.sync_copy(exstage, accum.at[dstv.at[ch]], add=True)
        return carry
    lax.fori_loop(0, _NCH_B, _chunk, 0)

    plsc.subcore_barrier()
    r0 = sid * _RPT

    @pl.when(cid == 0)
    def _():
        @pl.when(sid < 15)
        def _():
            pltpu.sync_copy(accum.at[pl.ds(r0, _RPT)], d0_hbm.at[pl.ds(r0, _RPT)])

        @pl.when(sid == 15)
        def _():
            pltpu.sync_copy(accum.at[pl.ds(15 * _RPT, 640)],
                            d0_hbm.at[pl.ds(15 * _RPT, 640)])

    @pl.when(cid == 1)
    def _():
        @pl.when(sid < 15)
        def _():
            pltpu.sync_copy(accum.at[pl.ds(r0, _RPT)], d1_hbm.at[pl.ds(r0, _RPT)])

        @pl.when(sid == 15)
        def _():
            pltpu.sync_copy(accum.at[pl.ds(15 * _RPT, 640)],
                            d1_hbm.at[pl.ds(15 * _RPT, 640)])


_edge_call = pl.kernel(
    _edge_body,
    out_type=[
        jax.ShapeDtypeStruct((_E, 16), jnp.float32),
        jax.ShapeDtypeStruct((_N, 16), jnp.float32),
        jax.ShapeDtypeStruct((_N, 16), jnp.float32),
    ],
    mesh=_mesh,
    scratch_types=[
        pltpu.VMEM((_NCH_B, _CH), jnp.int32),
        pltpu.VMEM((_NCH_B, _CH), jnp.int32),
        pltpu.VMEM((2, _CH, 512), jnp.int32),
        pltpu.VMEM((2, _CH, 512), jnp.int32),
        pltpu.VMEM((_CH, 16), jnp.float32),
        pltpu.VMEM((640, 16), jnp.float32),
        pltpu.VMEM_SHARED((_N, 16), jnp.float32),
        pltpu.SemaphoreType.DMA((2,)),
        pltpu.SemaphoreType.DMA((2,)),
    ],
    compiler_params=_sc_params,
)


def _agg_body(ex_hbm, dinv_hbm, va_hbm, vb_hbm, dst2d_hbm, src2d_hbm,
              pa_hbm, pb_hbm,
              dstv, srcv, vrows2, exrows, divrows2, urows2,
              zbuf, accum, semv, semd):
    cid = lax.axis_index("c")
    sid = lax.axis_index("s")

    pltpu.sync_copy(dst2d_hbm.at[sid], dstv)
    pltpu.sync_copy(src2d_hbm.at[sid], srcv)

    zero16 = jnp.zeros((16,), jnp.float32)

    def _z(r, carry):
        for j in range(4):
            zbuf[r, pl.ds(j * 16, 16)] = zero16
        return carry
    lax.fori_loop(0, 312, _z, 0)
    for j in range(2):
        pltpu.sync_copy(zbuf, accum.at[pl.ds(sid * _RPT + j * 312, 312)])

    @pl.when(sid == 15)
    def _():
        pltpu.sync_copy(zbuf.at[pl.ds(0, 16)], accum.at[pl.ds(9984, 16)])
    plsc.subcore_barrier()

    ebase = sid * _EPT_C

    def _issue(ch):
        par = lax.rem(ch, 2)

        @pl.when(cid == 0)
        def _():
            pltpu.async_copy(va_hbm.at[srcv.at[ch]], vrows2.at[par],
                             semv.at[par])

        @pl.when(cid == 1)
        def _():
            pltpu.async_copy(vb_hbm.at[srcv.at[ch]], vrows2.at[par],
                             semv.at[par])
        pltpu.async_copy(dinv_hbm.at[dstv.at[ch]], divrows2.at[par],
                         semd.at[par])

    _issue(0)

    def _chunk(ch, carry):
        par = lax.rem(ch, 2)
        pltpu.sync_copy(ex_hbm.at[pl.ds(ebase + ch * _CHA, _CHA)], exrows)
        pltpu.make_async_copy(va_hbm.at[srcv.at[ch]], vrows2.at[par],
                              semv.at[par]).wait()
        pltpu.make_async_copy(dinv_hbm.at[dstv.at[ch]], divrows2.at[par],
                              semd.at[par]).wait()

        @pl.when(ch < _NCHA - 1)
        def _():
            _issue(ch + 1)

        def _edge(i, c2):
            c = exrows[i] * divrows2[par, i]
            for j in range(4):
                u = None
                for h in range(_H):
                    seg = vrows2[par, i, pl.ds(h * 64 + j * 16, 16)] * c[h]
                    u = seg if u is None else u + seg
                urows2[par, i, pl.ds(j * 16, 16)] = u
            return c2
        lax.fori_loop(0, _CHA, _edge, 0)

        pltpu.sync_copy(urows2.at[par], accum.at[dstv.at[ch]], add=True)
        return carry
    lax.fori_loop(0, _NCHA, _chunk, 0)

    plsc.subcore_barrier()
    r0 = sid * _RPT

    @pl.when(cid == 0)
    def _():
        @pl.when(sid < 15)
        def _():
            pltpu.sync_copy(accum.at[pl.ds(r0, _RPT)], pa_hbm.at[pl.ds(r0, _RPT)])

        @pl.when(sid == 15)
        def _():
            pltpu.sync_copy(accum.at[pl.ds(15 * _RPT, 640)],
                            pa_hbm.at[pl.ds(15 * _RPT, 640)])

    @pl.when(cid == 1)
    def _():
        @pl.when(sid < 15)
        def _():
            pltpu.sync_copy(accum.at[pl.ds(r0, _RPT)], pb_hbm.at[pl.ds(r0, _RPT)])

        @pl.when(sid == 15)
        def _():
            pltpu.sync_copy(accum.at[pl.ds(15 * _RPT, 640)],
                            pb_hbm.at[pl.ds(15 * _RPT, 640)])


_agg_call = pl.kernel(
    _agg_body,
    out_type=[
        jax.ShapeDtypeStruct((_N, 64), jnp.float32),
        jax.ShapeDtypeStruct((_N, 64), jnp.float32),
    ],
    mesh=_mesh,
    scratch_types=[
        pltpu.VMEM((_NCHA, _CHA), jnp.int32),
        pltpu.VMEM((_NCHA, _CHA), jnp.int32),
        pltpu.VMEM((2, _CHA, 256), jnp.float32),
        pltpu.VMEM((_CHA, 16), jnp.float32),
        pltpu.VMEM((2, _CHA, 16), jnp.float32),
        pltpu.VMEM((2, _CHA, 64), jnp.float32),
        pltpu.VMEM((312, 64), jnp.float32),
        pltpu.VMEM_SHARED((_N, 64), jnp.float32),
        pltpu.SemaphoreType.DMA((2,)),
        pltpu.SemaphoreType.DMA((2,)),
    ],
    compiler_params=_sc_params,
)


def _dinv_body(d0_ref, d1_ref, o_ref):
    o_ref[...] = 0.25 / jnp.maximum(d0_ref[...] + d1_ref[...], 1e-16)


_dinv = pl.pallas_call(
    _dinv_body,
    grid=(10,),
    in_specs=[
        pl.BlockSpec((1000, 16), lambda i: (i, 0)),
        pl.BlockSpec((1000, 16), lambda i: (i, 0)),
    ],
    out_specs=pl.BlockSpec((1000, 16), lambda i: (i, 0)),
    out_shape=jax.ShapeDtypeStruct((_N, 16), jnp.float32),
)


def _final_body(pre_ref, sk_ref, o_ref):
    o_ref[...] = pre_ref[...] + sk_ref[...]


_final = pl.pallas_call(
    _final_body,
    grid=(10,),
    in_specs=[
        pl.BlockSpec((1000, 256), lambda i: (i, 0)),
        pl.BlockSpec((1000, 256), lambda i: (i, 0)),
    ],
    out_specs=pl.BlockSpec((1000, 256), lambda i: (i, 0)),
    out_shape=jax.ShapeDtypeStruct((_N, 256), jnp.float32),
)


def kernel(x, edge_indices, Wq, bq, Wk, bk, Wv, bv, Wskip, bskip):
    ei = edge_indices.astype(jnp.int32)
    srcB = ei[0].reshape(_NW, _NCH_B, _CH)
    dstB = ei[1].reshape(_NW, _NCH_B, _CH)
    srcC = ei[0].reshape(16, _NCHA, _CHA)
    dstC = ei[1].reshape(16, _NCHA, _CHA)

    Wv_r = Wv.reshape(_D, _H, 4, 64)
    bv_r = bv.reshape(_H, 4, 64)
    WvQ = [Wv_r[:, :, q, :].reshape(_D, 256) for q in range(4)]
    bvQ = [bv_r[:, q, :].reshape(256) for q in range(4)]
    W_all = jnp.concatenate([Wq, Wk] + WvQ + [Wskip], axis=1)
    b_all = jnp.concatenate([bq, bk] + bvQ + [bskip]).reshape(1, 3328)

    q, k, v0, v1, v2, v3, skip = _proj(x, W_all, b_all)
    q_i = lax.bitcast_convert_type(q.reshape(_N, 512, 2), jnp.int32)
    k_i = lax.bitcast_convert_type(k.reshape(_N, 512, 2), jnp.int32)
    ex, d0, d1 = _edge_call(q_i, k_i, dstB, srcB)
    dinv = _dinv(d0, d1)
    p0, p1 = _agg_call(ex, dinv, v0, v1, dstC, srcC)
    p2, p3 = _agg_call(ex, dinv, v2, v3, dstC, srcC)
    pre = jnp.concatenate([p0, p1, p2, p3], axis=1)
    return _final(pre, skip)


# speedup vs baseline: 1.4650x; 1.4650x over previous
import functools

import jax
import jax.numpy as jnp
from jax import lax
from jax.experimental import pallas as pl
from jax.experimental.pallas import tpu as pltpu
from jax.experimental.pallas import tpu_sc as plsc

_N, _E, _D, _H = 10000, 160000, 256, 4
_HC = 1024
_CH = 40
_NW = 32
_EPW_B = _E // _NW
_NCH_B = _EPW_B // _CH
_EPT_C = _E // 16
_CHA = 40
_NCHA = _EPT_C // _CHA
_RPT = 624

_mesh = plsc.VectorSubcoreMesh(core_axis_name="c", subcore_axis_name="s")
_sc_params = pltpu.CompilerParams(use_tc_tiling_on_sc=False,
                                  needs_layout_passes=False)


def _pack_bf16_pairs(z):
    b = lax.bitcast_convert_type(z, jnp.int32) + 0x8000
    lo = jnp.concatenate([b[:, h * 256:h * 256 + 128] for h in range(_H)],
                         axis=1)
    hi = jnp.concatenate([b[:, h * 256 + 128:(h + 1) * 256] for h in range(_H)],
                         axis=1)
    return (hi & jnp.int32(-65536)) | lax.shift_right_logical(lo, 16)


def _proj_body(x_ref, w_ref, b_ref, q_ref, k_ref,
               v0_ref, v1_ref, v2_ref, v3_ref, sk_ref):
    y = jnp.dot(x_ref[...], w_ref[...], preferred_element_type=jnp.float32)
    y = y + b_ref[...]
    q_ref[...] = _pack_bf16_pairs(y[:, :1024])
    k_ref[...] = _pack_bf16_pairs(y[:, 1024:2048])
    v0_ref[...] = y[:, 2048:2304]
    v1_ref[...] = y[:, 2304:2560]
    v2_ref[...] = y[:, 2560:2816]
    v3_ref[...] = y[:, 2816:3072]
    sk_ref[...] = y[:, 3072:]


_proj = pl.pallas_call(
    _proj_body,
    grid=(10,),
    in_specs=[
        pl.BlockSpec((1000, _D), lambda i: (i, 0)),
        pl.BlockSpec((_D, 3328), lambda i: (0, 0)),
        pl.BlockSpec((1, 3328), lambda i: (0, 0)),
    ],
    out_specs=[
        pl.BlockSpec((1000, 512), lambda i: (i, 0)),
        pl.BlockSpec((1000, 512), lambda i: (i, 0)),
        pl.BlockSpec((1000, 256), lambda i: (i, 0)),
        pl.BlockSpec((1000, 256), lambda i: (i, 0)),
        pl.BlockSpec((1000, 256), lambda i: (i, 0)),
        pl.BlockSpec((1000, 256), lambda i: (i, 0)),
        pl.BlockSpec((1000, 256), lambda i: (i, 0)),
    ],
    out_shape=[
        jax.ShapeDtypeStruct((_N, 512), jnp.int32),
        jax.ShapeDtypeStruct((_N, 512), jnp.int32),
        jax.ShapeDtypeStruct((_N, 256), jnp.float32),
        jax.ShapeDtypeStruct((_N, 256), jnp.float32),
        jax.ShapeDtypeStruct((_N, 256), jnp.float32),
        jax.ShapeDtypeStruct((_N, 256), jnp.float32),
        jax.ShapeDtypeStruct((_N, 256), jnp.float32),
    ],
)


def _edge_body(q_hbm, k_hbm, dst2d_hbm, src2d_hbm,
               ex_hbm, d0_hbm, d1_hbm,
               dstv, srcv, qrows, krows, exstage, zbuf, accum, sem1, sem2):
    cid = lax.axis_index("c")
    sid = lax.axis_index("s")
    wid = sid * 2 + cid

    pltpu.sync_copy(dst2d_hbm.at[wid], dstv)
    pltpu.sync_copy(src2d_hbm.at[wid], srcv)

    zero16 = jnp.zeros((16,), jnp.float32)

    def _z(r, carry):
        zbuf[r] = zero16
        return carry
    lax.fori_loop(0, 640, _z, 0)

    @pl.when(sid < 15)
    def _():
        pltpu.sync_copy(zbuf.at[pl.ds(0, _RPT)], accum.at[pl.ds(sid * _RPT, _RPT)])

    @pl.when(sid == 15)
    def _():
        pltpu.sync_copy(zbuf, accum.at[pl.ds(15 * _RPT, 640)])
    plsc.subcore_barrier()

    lanes = lax.iota(jnp.int32, 16)

    dnums = lax.GatherDimensionNumbers(
        offset_dims=(), collapsed_slice_dims=(0,), start_index_map=(0,))

    def _lane_sum(a):
        for s in (8, 4, 2, 1):
            perm = lax.gather(a, (lanes ^ s)[:, None], dnums, (1,),
                              mode=lax.GatherScatterMode.PROMISE_IN_BOUNDS)
            a = a + perm
        return a

    mask_hi = jnp.full((16,), -65536, jnp.int32)

    def _bf2f32(x):
        hi = plsc.bitcast(x & mask_hi, jnp.float32)
        lo = plsc.bitcast(lax.shift_left(x, 16), jnp.float32)
        return lo, hi

    def _issue(ch):
        par = lax.rem(ch, 2)
        pltpu.async_copy(q_hbm.at[dstv.at[ch]], qrows.at[par], sem1.at[par])
        pltpu.async_copy(k_hbm.at[srcv.at[ch]], krows.at[par], sem2.at[par])

    _issue(0)

    def _chunk(ch, carry):
        par = lax.rem(ch, 2)
        pltpu.make_async_copy(q_hbm.at[dstv.at[ch]], qrows.at[par],
                              sem1.at[par]).wait()
        pltpu.make_async_copy(k_hbm.at[srcv.at[ch]], krows.at[par],
                              sem2.at[par]).wait()

        @pl.when(ch < _NCH_B - 1)
        def _():
            _issue(ch + 1)

        def _edge(i, c2):
            accs = []
            for h in range(_H):
                a = None
                for t in range(8):
                    off = h * 128 + t * 16
                    qw = qrows[par, i, pl.ds(off, 16)]
                    kw = krows[par, i, pl.ds(off, 16)]
                    qa, qb = _bf2f32(qw)
                    ka, kb = _bf2f32(kw)
                    p = qa * ka + qb * kb
                    a = p if a is None else a + p
                accs.append(_lane_sum(a))
            lv = jnp.where(lanes == 0, accs[0],
                           jnp.where(lanes == 1, accs[1],
                                     jnp.where(lanes == 2, accs[2], accs[3])))
            ex = jnp.exp(lv * 0.0625)
            ex = jnp.where(lanes < _H, ex, 0.0)
            exstage[i] = ex
            return c2
        lax.fori_loop(0, _CH, _edge, 0)

        pltpu.sync_copy(exstage, ex_hbm.at[pl.ds(wid * _EPW_B + ch * _CH, _CH)])
        pltpu.sync_copy(exstage, accum.at[dstv.at[ch]], add=True)
        return carry
    lax.fori_loop(0, _NCH_B, _chunk, 0)

    plsc.subcore_barrier()
    r0 = sid * _RPT

    @pl.when(cid == 0)
    def _():
        @pl.when(sid < 15)
        def _():
            pltpu.sync_copy(accum.at[pl.ds(r0, _RPT)], d0_hbm.at[pl.ds(r0, _RPT)])

        @pl.when(sid == 15)
        def _():
            pltpu.sync_copy(accum.at[pl.ds(15 * _RPT, 640)],
                            d0_hbm.at[pl.ds(15 * _RPT, 640)])

    @pl.when(cid == 1)
    def _():
        @pl.when(sid < 15)
        def _():
            pltpu.sync_copy(accum.at[pl.ds(r0, _RPT)], d1_hbm.at[pl.ds(r0, _RPT)])

        @pl.when(sid == 15)
        def _():
            pltpu.sync_copy(accum.at[pl.ds(15 * _RPT, 640)],
                            d1_hbm.at[pl.ds(15 * _RPT, 640)])


_edge_call = pl.kernel(
    _edge_body,
    out_type=[
        jax.ShapeDtypeStruct((_E, 16), jnp.float32),
        jax.ShapeDtypeStruct((_N, 16), jnp.float32),
        jax.ShapeDtypeStruct((_N, 16), jnp.float32),
    ],
    mesh=_mesh,
    scratch_types=[
        pltpu.VMEM((_NCH_B, _CH), jnp.int32),
        pltpu.VMEM((_NCH_B, _CH), jnp.int32),
        pltpu.VMEM((2, _CH, 512), jnp.int32),
        pltpu.VMEM((2, _CH, 512), jnp.int32),
        pltpu.VMEM((_CH, 16), jnp.float32),
        pltpu.VMEM((640, 16), jnp.float32),
        pltpu.VMEM_SHARED((_N, 16), jnp.float32),
        pltpu.SemaphoreType.DMA((2,)),
        pltpu.SemaphoreType.DMA((2,)),
    ],
    compiler_params=_sc_params,
)


def _agg_body(ex_hbm, dinv_hbm, va_hbm, vb_hbm, dst2d_hbm, src2d_hbm,
              pa_hbm, pb_hbm,
              dstv, srcv, vrows2, exrows, divrows2, urows2,
              zbuf, accum, semv, semd):
    cid = lax.axis_index("c")
    sid = lax.axis_index("s")

    pltpu.sync_copy(dst2d_hbm.at[sid], dstv)
    pltpu.sync_copy(src2d_hbm.at[sid], srcv)

    zero16 = jnp.zeros((16,), jnp.float32)

    def _z(r, carry):
        for j in range(4):
            zbuf[r, pl.ds(j * 16, 16)] = zero16
        return carry
    lax.fori_loop(0, 312, _z, 0)
    for j in range(2):
        pltpu.sync_copy(zbuf, accum.at[pl.ds(sid * _RPT + j * 312, 312)])

    @pl.when(sid == 15)
    def _():
        pltpu.sync_copy(zbuf.at[pl.ds(0, 16)], accum.at[pl.ds(9984, 16)])
    plsc.subcore_barrier()

    ebase = sid * _EPT_C

    def _issue(ch):
        par = lax.rem(ch, 2)

        @pl.when(cid == 0)
        def _():
            pltpu.async_copy(va_hbm.at[srcv.at[ch]], vrows2.at[par],
                             semv.at[par])

        @pl.when(cid == 1)
        def _():
            pltpu.async_copy(vb_hbm.at[srcv.at[ch]], vrows2.at[par],
                             semv.at[par])
        pltpu.async_copy(dinv_hbm.at[dstv.at[ch]], divrows2.at[par],
                         semd.at[par])

    _issue(0)

    def _chunk(ch, carry):
        par = lax.rem(ch, 2)
        pltpu.sync_copy(ex_hbm.at[pl.ds(ebase + ch * _CHA, _CHA)], exrows)
        pltpu.make_async_copy(va_hbm.at[srcv.at[ch]], vrows2.at[par],
                              semv.at[par]).wait()
        pltpu.make_async_copy(dinv_hbm.at[dstv.at[ch]], divrows2.at[par],
                              semd.at[par]).wait()

        @pl.when(ch < _NCHA - 1)
        def _():
            _issue(ch + 1)

        def _edge(i, c2):
            c = exrows[i] * divrows2[par, i]
            for j in range(4):
                u = None
                for h in range(_H):
                    seg = vrows2[par, i, pl.ds(h * 64 + j * 16, 16)] * c[h]
                    u = seg if u is None else u + seg
                urows2[par, i, pl.ds(j * 16, 16)] = u
            return c2
        lax.fori_loop(0, _CHA, _edge, 0)

        pltpu.sync_copy(urows2.at[par], accum.at[dstv.at[ch]], add=True)
        return carry
    lax.fori_loop(0, _NCHA, _chunk, 0)

    plsc.subcore_barrier()
    r0 = sid * _RPT

    @pl.when(cid == 0)
    def _():
        @pl.when(sid < 15)
        def _():
            pltpu.sync_copy(accum.at[pl.ds(r0, _RPT)], pa_hbm.at[pl.ds(r0, _RPT)])

        @pl.when(sid == 15)
        def _():
            pltpu.sync_copy(accum.at[pl.ds(15 * _RPT, 640)],
                            pa_hbm.at[pl.ds(15 * _RPT, 640)])

    @pl.when(cid == 1)
    def _():
        @pl.when(sid < 15)
        def _():
            pltpu.sync_copy(accum.at[pl.ds(r0, _RPT)], pb_hbm.at[pl.ds(r0, _RPT)])

        @pl.when(sid == 15)
        def _():
            pltpu.sync_copy(accum.at[pl.ds(15 * _RPT, 640)],
                            pb_hbm.at[pl.ds(15 * _RPT, 640)])


_agg_call = pl.kernel(
    _agg_body,
    out_type=[
        jax.ShapeDtypeStruct((_N, 64), jnp.float32),
        jax.ShapeDtypeStruct((_N, 64), jnp.float32),
    ],
    mesh=_mesh,
    scratch_types=[
        pltpu.VMEM((_NCHA, _CHA), jnp.int32),
        pltpu.VMEM((_NCHA, _CHA), jnp.int32),
        pltpu.VMEM((2, _CHA, 256), jnp.float32),
        pltpu.VMEM((_CHA, 16), jnp.float32),
        pltpu.VMEM((2, _CHA, 16), jnp.float32),
        pltpu.VMEM((2, _CHA, 64), jnp.float32),
        pltpu.VMEM((312, 64), jnp.float32),
        pltpu.VMEM_SHARED((_N, 64), jnp.float32),
        pltpu.SemaphoreType.DMA((2,)),
        pltpu.SemaphoreType.DMA((2,)),
    ],
    compiler_params=_sc_params,
)


def _dinv_body(d0_ref, d1_ref, o_ref):
    o_ref[...] = 0.25 / jnp.maximum(d0_ref[...] + d1_ref[...], 1e-16)


_dinv = pl.pallas_call(
    _dinv_body,
    grid=(10,),
    in_specs=[
        pl.BlockSpec((1000, 16), lambda i: (i, 0)),
        pl.BlockSpec((1000, 16), lambda i: (i, 0)),
    ],
    out_specs=pl.BlockSpec((1000, 16), lambda i: (i, 0)),
    out_shape=jax.ShapeDtypeStruct((_N, 16), jnp.float32),
)


def _final_body(pre_ref, sk_ref, o_ref):
    o_ref[...] = pre_ref[...] + sk_ref[...]


_final = pl.pallas_call(
    _final_body,
    grid=(10,),
    in_specs=[
        pl.BlockSpec((1000, 256), lambda i: (i, 0)),
        pl.BlockSpec((1000, 256), lambda i: (i, 0)),
    ],
    out_specs=pl.BlockSpec((1000, 256), lambda i: (i, 0)),
    out_shape=jax.ShapeDtypeStruct((_N, 256), jnp.float32),
)


def kernel(x, edge_indices, Wq, bq, Wk, bk, Wv, bv, Wskip, bskip):
    ei = edge_indices.astype(jnp.int32)
    srcB = ei[0].reshape(_NW, _NCH_B, _CH)
    dstB = ei[1].reshape(_NW, _NCH_B, _CH)
    srcC = ei[0].reshape(16, _NCHA, _CHA)
    dstC = ei[1].reshape(16, _NCHA, _CHA)

    Wv_r = Wv.reshape(_D, _H, 4, 64)
    bv_r = bv.reshape(_H, 4, 64)
    WvQ = [Wv_r[:, :, q, :].reshape(_D, 256) for q in range(4)]
    bvQ = [bv_r[:, q, :].reshape(256) for q in range(4)]
    W_all = jnp.concatenate([Wq, Wk] + WvQ + [Wskip], axis=1)
    b_all = jnp.concatenate([bq, bk] + bvQ + [bskip]).reshape(1, 3328)

    q, k, v0, v1, v2, v3, skip = _proj(x, W_all, b_all)
    ex, d0, d1 = _edge_call(q, k, dstB, srcB)
    dinv = _dinv(d0, d1)
    p0, p1 = _agg_call(ex, dinv, v0, v1, dstC, srcC)
    p2, p3 = _agg_call(ex, dinv, v2, v3, dstC, srcC)
    pre = jnp.concatenate([p0, p1, p2, p3], axis=1)
    return _final(pre, skip)
